# baseline (speedup 1.0000x reference)
import functools

import jax
import jax.numpy as jnp
from jax import lax
from jax.experimental import pallas as pl
from jax.experimental.pallas import tpu as pltpu
from jax.experimental.pallas import tpu_sc as plsc

_B = 4096
_D = 32
_V = 1000000

_info = plsc.get_sparse_core_info()
_NC, _NS = _info.num_cores, _info.num_subcores
_NW = _NC * _NS
_BPW = _B // _NW
_G = 16


def _make_gather():
    n_groups = _BPW // _G
    mesh = plsc.VectorSubcoreMesh(core_axis_name="c", subcore_axis_name="s")

    @functools.partial(
        pl.kernel,
        mesh=mesh,
        compiler_params=pltpu.CompilerParams(
            use_tc_tiling_on_sc=True, needs_layout_passes=False),
        out_type=(
            jax.ShapeDtypeStruct((_B, _D), jnp.float32),
            jax.ShapeDtypeStruct((_B, _D), jnp.float32),
        ),
        scratch_types=[
            pltpu.VMEM((_BPW,), jnp.int32),
            pltpu.VMEM((_BPW,), jnp.int32),
            pltpu.VMEM((_BPW, _D), jnp.float32),
            pltpu.VMEM((_BPW, _D), jnp.float32),
            pltpu.SemaphoreType.DMA,
            pltpu.SemaphoreType.DMA,
        ],
    )
    def gather_k(tgt_hbm, ctx_hbm, in_hbm, out_hbm, a_out, b_out,
                 idx_a, idx_b, a_v, b_v, sem_a, sem_b):
        wid = lax.axis_index("s") * _NC + lax.axis_index("c")
        base = pl.multiple_of(wid * _BPW, 128)
        pltpu.sync_copy(tgt_hbm.at[pl.ds(base, _BPW)], idx_a)
        pltpu.sync_copy(ctx_hbm.at[pl.ds(base, _BPW)], idx_b)

        def phase(idx_ref, src_ref, dst_ref):
            def issue(vb, j, sem):
                pltpu.async_copy(
                    src_ref.at[pl.ds(vb, 1), :],
                    dst_ref.at[pl.ds(j, 1), :], sem)

            vv0 = idx_ref[pl.ds(0, _G)]
            for b in range(8):
                issue(vv0[b], b, sem_a)
            for b in range(8, 16):
                issue(vv0[b], 8 + (b - 8), sem_b)

            def group(g, vcur):
                nxt = jnp.minimum((g + 1) * _G, _BPW - _G)
                vnxt = idx_ref[pl.ds(nxt, _G)]
                not_last = g < n_groups - 1
                for half, sem in ((0, sem_a), (1, sem_b)):
                    for b in range(half * 8, half * 8 + 8):
                        pltpu.make_async_copy(
                            src_ref.at[pl.ds(0, 1), :],
                            dst_ref.at[pl.ds(0, 1), :], sem).wait()

                    @pl.when(not_last)
                    def _():
                        for b in range(half * 8, half * 8 + 8):
                            issue(vnxt[b], (g + 1) * _G + b, sem)
                return vnxt

            lax.fori_loop(0, n_groups, group, vv0)

        phase(idx_a, in_hbm, a_v)
        phase(idx_b, out_hbm, b_v)
        pltpu.sync_copy(a_v, a_out.at[pl.ds(base, _BPW), :])
        pltpu.sync_copy(b_v, b_out.at[pl.ds(base, _BPW), :])

    return gather_k


_gather = _make_gather()

_BM = 512
_NSTRIPE = 4
_SPS = _B // _NSTRIPE // _BM


def _mm(a_ref, b_ref, o_ref):
    o_ref[...] = lax.dot_general(
        a_ref[...], b_ref[...],
        (((1,), (1,)), ((), ())),
        preferred_element_type=jnp.float32,
    )


def _mm_prev(prev_ref, a_ref, b_ref, o_ref):
    del prev_ref
    _mm(a_ref, b_ref, o_ref)


@functools.cache
def _make_mm(stripe):
    row0 = stripe * _SPS
    ab_specs = [
        pl.BlockSpec((_BM, _D), lambda i: (row0 + i, 0)),
        pl.BlockSpec((_B, _D), lambda i: (0, 0)),
    ]
    out_spec = pl.BlockSpec((_BM, _B), lambda i: (row0 + i, 0))
    out_shape = jax.ShapeDtypeStruct((_B, _B), jnp.float32)
    if stripe == 0:
        return pl.pallas_call(
            _mm,
            grid=(_SPS,),
            in_specs=ab_specs,
            out_specs=out_spec,
            out_shape=out_shape,
        )
    return pl.pallas_call(
        _mm_prev,
        grid=(_SPS,),
        in_specs=[pl.BlockSpec(memory_space=pl.ANY)] + ab_specs,
        out_specs=out_spec,
        out_shape=out_shape,
        input_output_aliases={0: 0},
    )


def kernel(target, context, in_embed, out_embed):
    a, b = _gather(
        target.astype(jnp.int32), context.astype(jnp.int32),
        in_embed, out_embed,
    )
    scores = _make_mm(0)(a, b)
    for i in range(1, _NSTRIPE):
        scores = _make_mm(i)(scores, a, b)
    return scores
